# baseline (speedup 1.0000x reference)
import functools

import jax
import jax.numpy as jnp
from jax import lax
from jax.experimental import pallas as pl
from jax.experimental.pallas import tpu as pltpu
from jax.experimental.pallas import tpu_sc as plsc

_VOCAB = 1000000
_FDIM = 64
_EDIM = 1024
_MAXLEN = 2048

_NC = 2
_NS = 16
_NW = _NC * _NS
_CHUNK = 128


def _sc_gather(idx2d, table, n_rows):
    n_chunks = n_rows // _CHUNK
    chunks_per_w = n_chunks // _NW
    rows_per_w = n_rows // _NW
    mesh = plsc.VectorSubcoreMesh(core_axis_name="c", subcore_axis_name="s")

    @functools.partial(
        pl.kernel,
        mesh=mesh,
        out_type=jax.ShapeDtypeStruct((n_rows, _FDIM), jnp.float32),
        scratch_types=[
            pltpu.VMEM((chunks_per_w, _CHUNK), jnp.int32),
            pltpu.VMEM((rows_per_w, _FDIM), jnp.float32),
            pltpu.SemaphoreType.DMA,
        ],
        compiler_params=pltpu.CompilerParams(use_tc_tiling_on_sc=False),
    )
    def gather_kernel(idx_hbm, table_hbm, out_hbm, idx_v, rows_v, sem):
        wid = lax.axis_index("s") * _NC + lax.axis_index("c")
        pltpu.sync_copy(idx_hbm.at[pl.ds(wid * chunks_per_w, chunks_per_w)], idx_v)
        copies = []
        for j in range(chunks_per_w):
            cp = pltpu.make_async_copy(
                table_hbm.at[idx_v.at[j]],
                rows_v.at[pl.ds(j * _CHUNK, _CHUNK)],
                sem,
            )
            cp.start()
            copies.append(cp)
        for cp in copies:
            cp.wait()
        pltpu.sync_copy(rows_v, out_hbm.at[pl.ds(wid * rows_per_w, rows_per_w)])

    return gather_kernel(idx2d, table)


def _tc_project_add(tok_low, factorized_table, pos_table, b, l):
    blk = 256
    l_blocks = l // blk

    def body(tok_ref, fac_ref, pos_ref, out_ref):
        out_ref[...] = (
            jnp.dot(tok_ref[...], fac_ref[...], preferred_element_type=jnp.float32)
            + pos_ref[...]
        )

    out = pl.pallas_call(
        body,
        grid=(l_blocks, b),
        in_specs=[
            pl.BlockSpec((blk, _FDIM), lambda i, j: (j * l_blocks + i, 0)),
            pl.BlockSpec((_FDIM, _EDIM), lambda i, j: (0, 0)),
            pl.BlockSpec((blk, _EDIM), lambda i, j: (i, 0)),
        ],
        out_specs=pl.BlockSpec((blk, _EDIM), lambda i, j: (j * l_blocks + i, 0)),
        out_shape=jax.ShapeDtypeStruct((b * l, _EDIM), jnp.float32),
    )(tok_low, factorized_table, pos_table)
    return out


def kernel(inputs, token_table, factorized_table, segment_table, pos_table):
    b, l = inputs.shape
    n_rows = b * l
    idx2d = inputs.astype(jnp.int32).reshape(n_rows // _CHUNK, _CHUNK)
    tok_low = _sc_gather(idx2d, token_table, n_rows)
    out = _tc_project_add(tok_low, factorized_table, pos_table, b, l)
    return out.reshape(b, l, _EDIM)
